# baseline (speedup 1.0000x reference)
import functools
import jax, jax.numpy as jnp, numpy as np
from jax import lax
from jax.experimental import pallas as pl
from jax.experimental.pallas import tpu as pltpu
from jax.experimental.pallas import tpu_sc as plsc

N, D, B = 320000, 128, 2500
NW = 32
Bp = 2560
SPW = Bp // NW

mesh = plsc.VectorSubcoreMesh(core_axis_name="c", subcore_axis_name="s")


@functools.partial(
    pl.kernel, mesh=mesh,
    out_type=jax.ShapeDtypeStruct((Bp, D), jnp.float32),
    scratch_types=[
        pltpu.VMEM((SPW + 16,), jnp.int32),
        pltpu.VMEM((SPW + 16,), jnp.int32),
        pltpu.VMEM((1, D), jnp.float32),
        pltpu.VMEM((SPW, D), jnp.float32),
        pltpu.SemaphoreType.DMA,
    ],
)
def sc_agg(x_hbm, starts_hbm, ends_hbm, out_hbm, st_v, en_v, row_v, acc_v,
           sem):
    wid = lax.axis_index("s") * 2 + lax.axis_index("c")
    base = wid * SPW
    pltpu.sync_copy(starts_hbm.at[pl.ds(base, SPW)], st_v.at[pl.ds(0, SPW)])
    pltpu.sync_copy(ends_hbm.at[pl.ds(base, SPW)], en_v.at[pl.ds(0, SPW)])

    zero = jnp.zeros((16,), jnp.float32)

    def seg_body(j, carry):
        r0 = st_v[pl.ds(j, 16)][0]
        r1 = en_v[pl.ds(j, 16)][0]
        for k in range(D // 16):
            acc_v[j, pl.ds(k * 16, 16)] = zero

        def row_body(r, c):
            cp = pltpu.make_async_copy(x_hbm.at[pl.ds(r, 1), :], row_v, sem)
            cp.start()
            cp.wait()
            for k in range(D // 16):
                acc_v[j, pl.ds(k * 16, 16)] = (
                    acc_v[j, pl.ds(k * 16, 16)] + row_v[0, pl.ds(k * 16, 16)])
            return c

        lax.fori_loop(r0, r1, row_body, 0)
        return carry

    lax.fori_loop(0, SPW, seg_body, 0)
    pltpu.sync_copy(acc_v, out_hbm.at[pl.ds(base, SPW)])


def _mlp_kern(x_ref, wi_ref, bi_ref, wo_ref, bo_ref, out_ref):
    agg = x_ref[...]
    h = lax.dot_general(agg, wi_ref[...], (((1,), (1,)), ((), ())),
                        preferred_element_type=jnp.float32) + bi_ref[...]
    u = jnp.exp(h)
    v = u * (u + 2.0)
    h = jnp.where(h > 20.0, h, h * (v / (v + 2.0)))
    out_ref[...] = lax.dot_general(
        h, wo_ref[...], (((1,), (1,)), ((), ())),
        preferred_element_type=jnp.float32) + bo_ref[...]


@jax.jit
def hybrid(node_embeddings, node_sizes, W_inner, b_inner, W_outer, b_outer):
    sizes = node_sizes.astype(jnp.int32)
    ends = jnp.cumsum(sizes)
    starts = ends - sizes
    pad = jnp.full((Bp - B,), ends[-1], jnp.int32)
    ends_p = jnp.concatenate([ends, pad])
    starts_p = jnp.concatenate([starts, pad])
    agg = sc_agg(node_embeddings, starts_p, ends_p)
    out = pl.pallas_call(
        _mlp_kern,
        out_shape=jax.ShapeDtypeStruct((B, W_outer.shape[0]), jnp.float32),
        in_specs=[pl.BlockSpec(memory_space=pltpu.VMEM)] * 5,
        out_specs=pl.BlockSpec(memory_space=pltpu.VMEM),
    )(agg[:B], W_inner, b_inner.reshape(1, -1), W_outer,
      b_outer.reshape(1, -1))
    return out


def kernel(node_embeddings, node_sizes, W_inner, b_inner, W_outer, b_outer):
    return hybrid(node_embeddings, node_sizes, W_inner, b_inner, W_outer,
                  b_outer)


# speedup vs baseline: 6.2174x; 6.2174x over previous
import functools

import jax
import jax.numpy as jnp
from jax import lax
from jax.experimental import pallas as pl
from jax.experimental.pallas import tpu as pltpu

_ST = 128
_WR = 128
_RC = 512
_NBUF = 4


def _make_kern(N):
    def _kern(x_hbm, sizes_ref, r0s_smem, r1s_smem, wi_ref, bi_ref, wo_ref,
              bo_ref, out_ref, xbuf, cbuf, acc_ref, sem, csem):
        G, L = sizes_ref.shape
        Bp = G * L
        T = Bp // _ST
        B = out_ref.shape[0]

        sizes = sizes_ref[...]
        ii = lax.broadcasted_iota(jnp.int32, (L, L), 0)
        jj = lax.broadcasted_iota(jnp.int32, (L, L), 1)
        triu = jnp.where(ii <= jj, 1.0, 0.0)
        ident = jnp.where(ii == jj, 1.0, 0.0)
        gi = lax.broadcasted_iota(jnp.int32, (G, G), 0)
        gj = lax.broadcasted_iota(jnp.int32, (G, G), 1)
        lstrict = jnp.where(gj < gi, 1.0, 0.0)
        within = lax.dot_general(sizes, triu, (((1,), (0,)), ((), ())),
                                 precision=lax.Precision.HIGHEST,
                                 preferred_element_type=jnp.float32)
        tot = within[:, L - 1:L]
        off = lax.dot_general(lstrict, tot, (((1,), (0,)), ((), ())),
                              precision=lax.Precision.HIGHEST,
                              preferred_element_type=jnp.float32)
        ends_g = within + off
        starts_g = ends_g - sizes
        ends_t = lax.dot_general(ident, ends_g, (((1,), (1,)), ((), ())),
                                 precision=lax.Precision.HIGHEST,
                                 preferred_element_type=jnp.float32)
        starts_t = lax.dot_general(ident, starts_g, (((1,), (1,)), ((), ())),
                                     precision=lax.Precision.HIGHEST,
                                     preferred_element_type=jnp.float32)
        ends_ti = ends_t.astype(jnp.int32)
        starts_ti = starts_t.astype(jnp.int32)

        acc_ref[...] = jnp.zeros_like(acc_ref)
        io = lax.broadcasted_iota(jnp.int32, (_ST, _WR), 1)
        ioc = lax.broadcasted_iota(jnp.int32, (_ST, _RC), 1)

        def r0_of(t):
            return r0s_smem[t]

        def r1_of(t):
            return r1s_smem[t]

        def win_copy(t):
            d0 = jnp.minimum(r0_of(t), N - _WR)
            return pltpu.make_async_copy(
                x_hbm.at[pl.ds(d0, _WR), :], xbuf.at[t % _NBUF],
                sem.at[t % _NBUF])

        for t in range(min(_NBUF, T)):
            @pl.when(r1_of(t) > r0_of(t))
            def _(t=t):
                win_copy(t).start()

        for t in range(T):
            @pl.when(r1_of(t) > r0_of(t))
            def _(t=t):
                win_copy(t).wait()
                d0 = jnp.minimum(r0_of(t), N - _WR)
                st = starts_ti[:, t:t + 1]
                en = ends_ti[:, t:t + 1]
                r = io + d0
                m = jnp.where((r >= st) & (r < en), 1.0, 0.0)
                acc_ref[t * _ST:(t + 1) * _ST, :] += lax.dot_general(
                    m, xbuf[t % _NBUF], (((1,), (0,)), ((), ())),
                    preferred_element_type=jnp.float32)

            if t + _NBUF < T:
                @pl.when(r1_of(t + _NBUF) > r0_of(t + _NBUF))
                def _(t=t):
                    win_copy(t + _NBUF).start()

        for t in range(T):
            extra = r1_of(t) - r0_of(t) - _WR
            trip = lax.div(jnp.maximum(extra, 0) + (_RC - 1), _RC)
            st = starts_ti[:, t:t + 1]
            en = ends_ti[:, t:t + 1]

            def body(j, carry, t=t, st=st, en=en):
                rr0 = r0_of(t) + _WR + j * _RC
                d0 = jnp.minimum(rr0, N - _RC)
                cp = pltpu.make_async_copy(
                    x_hbm.at[pl.ds(d0, _RC), :], cbuf, csem)
                cp.start()
                cp.wait()
                r = ioc + d0
                m = jnp.where((r >= st) & (r < en) & (r >= rr0), 1.0, 0.0)
                acc_ref[t * _ST:(t + 1) * _ST, :] += lax.dot_general(
                    m, cbuf[...], (((1,), (0,)), ((), ())),
                    preferred_element_type=jnp.float32)
                return carry

            lax.fori_loop(0, trip, body, 0)

        agg = acc_ref[...]
        h = lax.dot_general(agg, wi_ref[...], (((1,), (1,)), ((), ())),
                            preferred_element_type=jnp.float32) + bi_ref[...]
        u = jnp.exp(h)
        v = u * (u + 2.0)
        h = jnp.where(h > 20.0, h, h * (v / (v + 2.0)))
        y = lax.dot_general(h, wo_ref[...], (((1,), (1,)), ((), ())),
                            preferred_element_type=jnp.float32) + bo_ref[...]
        out_ref[...] = y[:B, :]

    return _kern


@functools.partial(jax.jit, static_argnames=("interpret",))
def _sum_readout(node_embeddings, node_sizes, W_inner, b_inner, W_outer,
                 b_outer, interpret=False):
    N, d_in = node_embeddings.shape
    B = node_sizes.shape[0]
    d_out = W_outer.shape[0]
    Bp = ((B + 127) // 128) * 128
    G = Bp // 128

    sizes_f = jnp.pad(node_sizes.astype(jnp.float32),
                      (0, Bp - B)).reshape(G, 128)
    tile_tot = jnp.sum(sizes_f, axis=1).astype(jnp.int32)
    tile_end = jnp.cumsum(tile_tot)
    tile_start = tile_end - tile_tot

    out = pl.pallas_call(
        _make_kern(N),
        out_shape=jax.ShapeDtypeStruct((B, d_out), jnp.float32),
        in_specs=[
            pl.BlockSpec(memory_space=pl.ANY),
            pl.BlockSpec(memory_space=pltpu.VMEM),
            pl.BlockSpec(memory_space=pltpu.SMEM),
            pl.BlockSpec(memory_space=pltpu.SMEM),
            pl.BlockSpec(memory_space=pltpu.VMEM),
            pl.BlockSpec(memory_space=pltpu.VMEM),
            pl.BlockSpec(memory_space=pltpu.VMEM),
            pl.BlockSpec(memory_space=pltpu.VMEM),
        ],
        out_specs=pl.BlockSpec(memory_space=pltpu.VMEM),
        scratch_shapes=[
            pltpu.VMEM((_NBUF, _WR, d_in), jnp.float32),
            pltpu.VMEM((_RC, d_in), jnp.float32),
            pltpu.VMEM((Bp, d_in), jnp.float32),
            pltpu.SemaphoreType.DMA((_NBUF,)),
            pltpu.SemaphoreType.DMA,
        ],
        interpret=interpret,
    )(node_embeddings, sizes_f, tile_start, tile_end, W_inner,
      b_inner.reshape(1, -1), W_outer, b_outer.reshape(1, -1))
    return out


def kernel(node_embeddings, node_sizes, W_inner, b_inner, W_outer, b_outer):
    return _sum_readout(node_embeddings, node_sizes, W_inner, b_inner,
                        W_outer, b_outer)
